# baseline (speedup 1.0000x reference)
import jax
import jax.numpy as jnp
from jax.experimental import pallas as pl
from jax.experimental.pallas import tpu as pltpu


def _moe_body(n_ref, xt_ref, W1_ref, b1_ref, W2_ref, b2_ref, W3_ref, b3_ref,
              Wg_ref, bg_ref, loss_ref, idx_ref,
              featb, glog, mse_s):
    DT, J, B, D = xt_ref.shape
    IN = W1_ref.shape[2]
    K = n_ref.shape[1]
    NJ = pl.num_programs(0)
    NE = pl.num_programs(1)
    j = pl.program_id(0)
    e = pl.program_id(1)
    f32 = jnp.float32

    @pl.when(e == 0)
    def _build_feat():
        for t in range(DT):
            for k in range(K):
                n = n_ref[j, k]
                sl = xt_ref[t, n]
                s = (t * K + k) * D
                featb[:, s:s + D] = sl.astype(jnp.bfloat16)

    @pl.when(e == 0)
    def _gating():
        wg = Wg_ref[0].astype(jnp.bfloat16)
        glog[...] = (jnp.dot(featb[...], wg, preferred_element_type=f32)
                     + bg_ref[0])

    tgt = xt_ref[DT // 2, j]
    EB = W1_ref.shape[1]

    def _mlp_bf16(u):
        a = featb[...]
        w1 = W1_ref[0, u].astype(jnp.bfloat16)
        h = jnp.dot(a, w1, preferred_element_type=f32) + b1_ref[0, u]
        h = jnp.maximum(h, 0.0)
        w2 = W2_ref[0, u].astype(jnp.bfloat16)
        h2 = jnp.dot(h.astype(jnp.bfloat16), w2, preferred_element_type=f32) + b2_ref[0, u]
        h2 = jnp.maximum(h2, 0.0)
        w3 = W3_ref[0, u].astype(jnp.bfloat16)
        pred = jnp.dot(h2.astype(jnp.bfloat16), w3, preferred_element_type=f32) + b3_ref[0, u]
        d = pred - tgt
        mse_s[e * EB + u] = jnp.mean(d * d, axis=1)[None, :]

    for u in range(EB):
        _mlp_bf16(u)

    @pl.when(e == NE - 1)
    def _finish_joint():
        mm = mse_s[:, 0, :]
        gT = jnp.transpose(glog[...])
        gT = gT - jnp.max(gT, axis=0, keepdims=True)
        p = jnp.exp(gT)
        gate = p / jnp.sum(p, axis=0, keepdims=True)
        contrib = jnp.sum(gate * mm) / (B * NJ)

        @pl.when(j == 0)
        def _():
            loss_ref[0, 0] = contrib

        @pl.when(j > 0)
        def _():
            loss_ref[0, 0] += contrib

    @pl.when((e == NE - 1) & (j == NJ - 1))
    def _argmin_last_joint():
        mm = mse_s[:, 0, :]
        bv = mm[0:1, :]
        bi = jnp.zeros((1, B), jnp.int32)
        for k in range(1, mm.shape[0]):
            rk = mm[k:k + 1, :]
            take = rk < bv
            bi = jnp.where(take, k, bi)
            bv = jnp.where(take, rk, bv)
        idx_ref[...] = bi


def kernel(x, W1, b1, W2, b2, W3, b3, Wg, bg, neighbors):
    B, DT, J, D = x.shape
    _, E, IN, H = W1.shape
    DOUT = W3.shape[-1]

    x5 = x.reshape(B, DT, J, 1, D)

    def _tr_body(x_ref, o_ref):
        o_ref[0, 0] = x_ref[:, 0, 0, 0, :]

    xt = pl.pallas_call(
        _tr_body,
        grid=(DT, J),
        in_specs=[pl.BlockSpec((B, 1, 1, 1, D), lambda t, jj: (0, t, jj, 0, 0))],
        out_specs=pl.BlockSpec((1, 1, B, D), lambda t, jj: (t, jj, 0, 0)),
        out_shape=jax.ShapeDtypeStruct((DT, J, B, D), jnp.float32),
    )(x5)
    b1r = b1.reshape(J, E, 1, H)
    b2r = b2.reshape(J, E, 1, H)
    b3r = b3.reshape(J, E, 1, DOUT)
    bgr = bg.reshape(J, 1, E)

    def c00(j, e):
        return (0, 0)

    EB = 4

    loss2d, idx2d = pl.pallas_call(
        _moe_body,
        grid=(J, E // EB),
        in_specs=[
            pl.BlockSpec(memory_space=pltpu.SMEM),
            pl.BlockSpec((DT, J, B, D), lambda j, e: (0, 0, 0, 0)),
            pl.BlockSpec((1, EB, IN, H), lambda j, e: (j, e, 0, 0)),
            pl.BlockSpec((1, EB, 1, H), lambda j, e: (j, e, 0, 0)),
            pl.BlockSpec((1, EB, H, H), lambda j, e: (j, e, 0, 0)),
            pl.BlockSpec((1, EB, 1, H), lambda j, e: (j, e, 0, 0)),
            pl.BlockSpec((1, EB, H, DOUT), lambda j, e: (j, e, 0, 0)),
            pl.BlockSpec((1, EB, 1, DOUT), lambda j, e: (j, e, 0, 0)),
            pl.BlockSpec((1, IN, E), lambda j, e: (j, 0, 0)),
            pl.BlockSpec((1, 1, E), lambda j, e: (j, 0, 0)),
        ],
        out_specs=[
            pl.BlockSpec((1, 1), c00, memory_space=pltpu.SMEM),
            pl.BlockSpec((1, B), c00),
        ],
        out_shape=[
            jax.ShapeDtypeStruct((1, 1), jnp.float32),
            jax.ShapeDtypeStruct((1, B), jnp.int32),
        ],
        scratch_shapes=[
            pltpu.VMEM((B, IN), jnp.bfloat16),
            pltpu.VMEM((B, E), jnp.float32),
            pltpu.VMEM((E, 1, B), jnp.float32),
        ],
    )(neighbors, xt, W1, b1r, W2, b2r, W3, b3r, Wg, bgr)

    return loss2d[0, 0], idx2d[0]


# speedup vs baseline: 1.2064x; 1.2064x over previous
import jax
import jax.numpy as jnp
from jax.experimental import pallas as pl
from jax.experimental.pallas import tpu as pltpu


def _moe_body(n_ref, xt_ref, W1_ref, b1_ref, W2_ref, b2_ref, W3_ref, b3_ref,
              Wg_ref, bg_ref, loss_ref, idx_ref,
              featb, glog, mse_s):
    DT, J, B, D = xt_ref.shape
    IN = W1_ref.shape[2]
    K = n_ref.shape[1]
    NJ = pl.num_programs(0)
    NE = pl.num_programs(1)
    j = pl.program_id(0)
    e = pl.program_id(1)
    f32 = jnp.float32

    @pl.when(e == 0)
    def _build_feat():
        for t in range(DT):
            for k in range(K):
                n = n_ref[j, k]
                sl = xt_ref[t, n]
                s = (t * K + k) * D
                featb[:, s:s + D] = sl.astype(jnp.bfloat16)

    @pl.when(e == 0)
    def _gating():
        wg = Wg_ref[0].astype(jnp.bfloat16)
        glog[...] = (jnp.dot(featb[...], wg, preferred_element_type=f32)
                     + bg_ref[0])

    tgt = xt_ref[DT // 2, j]
    EB = W1_ref.shape[1]

    def _mlp_bf16(u):
        a = featb[...]
        w1 = W1_ref[0, u].astype(jnp.bfloat16)
        h = jnp.dot(a, w1, preferred_element_type=f32) + b1_ref[0, u]
        h = jnp.maximum(h, 0.0)
        w2 = W2_ref[0, u].astype(jnp.bfloat16)
        h2 = jnp.dot(h.astype(jnp.bfloat16), w2, preferred_element_type=f32) + b2_ref[0, u]
        h2 = jnp.maximum(h2, 0.0)
        w3 = W3_ref[0, u].astype(jnp.bfloat16)
        pred = jnp.dot(h2.astype(jnp.bfloat16), w3, preferred_element_type=f32) + b3_ref[0, u]
        d = pred - tgt
        mse_s[e * EB + u] = jnp.mean(d * d, axis=1)[None, :]

    for u in range(EB):
        _mlp_bf16(u)

    @pl.when(e == NE - 1)
    def _finish_joint():
        mm = mse_s[:, 0, :]
        gT = jnp.transpose(glog[...])
        gT = gT - jnp.max(gT, axis=0, keepdims=True)
        p = jnp.exp(gT)
        gate = p / jnp.sum(p, axis=0, keepdims=True)
        contrib = jnp.sum(gate * mm) / (B * NJ)

        @pl.when(j == 0)
        def _():
            loss_ref[0, 0] = contrib

        @pl.when(j > 0)
        def _():
            loss_ref[0, 0] += contrib

    @pl.when((e == NE - 1) & (j == NJ - 1))
    def _argmin_last_joint():
        mm = mse_s[:, 0, :]
        bv = mm[0:1, :]
        bi = jnp.zeros((1, B), jnp.int32)
        for k in range(1, mm.shape[0]):
            rk = mm[k:k + 1, :]
            take = rk < bv
            bi = jnp.where(take, k, bi)
            bv = jnp.where(take, rk, bv)
        idx_ref[...] = bi


def kernel(x, W1, b1, W2, b2, W3, b3, Wg, bg, neighbors):
    B, DT, J, D = x.shape
    _, E, IN, H = W1.shape
    DOUT = W3.shape[-1]

    x5 = x.reshape(B, DT, J, 1, D)

    def _tr_body(x_ref, o_ref):
        for jj in range(x_ref.shape[2]):
            o_ref[0, jj] = x_ref[:, 0, jj, 0, :]

    xt = pl.pallas_call(
        _tr_body,
        grid=(DT,),
        in_specs=[pl.BlockSpec((B, 1, J, 1, D), lambda t: (0, t, 0, 0, 0))],
        out_specs=pl.BlockSpec((1, J, B, D), lambda t: (t, 0, 0, 0)),
        out_shape=jax.ShapeDtypeStruct((DT, J, B, D), jnp.float32),
    )(x5)
    b1r = b1.reshape(J, E, 1, H)
    b2r = b2.reshape(J, E, 1, H)
    b3r = b3.reshape(J, E, 1, DOUT)
    bgr = bg.reshape(J, 1, E)

    def c00(j, e):
        return (0, 0)

    EB = 4

    loss2d, idx2d = pl.pallas_call(
        _moe_body,
        grid=(J, E // EB),
        in_specs=[
            pl.BlockSpec(memory_space=pltpu.SMEM),
            pl.BlockSpec((DT, J, B, D), lambda j, e: (0, 0, 0, 0)),
            pl.BlockSpec((1, EB, IN, H), lambda j, e: (j, e, 0, 0)),
            pl.BlockSpec((1, EB, 1, H), lambda j, e: (j, e, 0, 0)),
            pl.BlockSpec((1, EB, H, H), lambda j, e: (j, e, 0, 0)),
            pl.BlockSpec((1, EB, 1, H), lambda j, e: (j, e, 0, 0)),
            pl.BlockSpec((1, EB, H, DOUT), lambda j, e: (j, e, 0, 0)),
            pl.BlockSpec((1, EB, 1, DOUT), lambda j, e: (j, e, 0, 0)),
            pl.BlockSpec((1, IN, E), lambda j, e: (j, 0, 0)),
            pl.BlockSpec((1, 1, E), lambda j, e: (j, 0, 0)),
        ],
        out_specs=[
            pl.BlockSpec((1, 1), c00, memory_space=pltpu.SMEM),
            pl.BlockSpec((1, B), c00),
        ],
        out_shape=[
            jax.ShapeDtypeStruct((1, 1), jnp.float32),
            jax.ShapeDtypeStruct((1, B), jnp.int32),
        ],
        scratch_shapes=[
            pltpu.VMEM((B, IN), jnp.bfloat16),
            pltpu.VMEM((B, E), jnp.float32),
            pltpu.VMEM((E, 1, B), jnp.float32),
        ],
    )(neighbors, xt, W1, b1r, W2, b2r, W3, b3r, Wg, bgr)

    return loss2d[0, 0], idx2d[0]
